# Initial kernel scaffold; baseline (speedup 1.0000x reference)
import functools

import jax
import jax.numpy as jnp
from jax import lax
from jax.experimental import pallas as pl
from jax.experimental.pallas import tpu as pltpu
from jax.experimental.pallas import tpu_sc as plsc

VOCAB = 100000
D = 200
B_TOTAL = 4096 * 50

NC = 2
NS = 16
NW = NC * NS

CHUNK = 128
PER_W = B_TOTAL // NW
NCH = PER_W // CHUNK


def _gather_body(idx_hbm, table_hbm, out_hbm, idx_v, rows_v, sem):
    wid = lax.axis_index("s") * NC + lax.axis_index("c")
    pltpu.sync_copy(idx_hbm.at[pl.ds(wid * NCH, NCH)], idx_v)
    out_base = wid * PER_W

    def step(j, carry):
        pltpu.async_copy(table_hbm.at[idx_v.at[j]], rows_v, sem).wait()
        pltpu.sync_copy(rows_v, out_hbm.at[pl.ds(out_base + j * CHUNK, CHUNK)])
        return carry

    lax.fori_loop(0, NCH, step, 0)


@jax.jit
def _embedding_gather(idx2d, table):
    return pl.kernel(
        _gather_body,
        out_type=jax.ShapeDtypeStruct((B_TOTAL, D), jnp.float32),
        mesh=plsc.VectorSubcoreMesh(core_axis_name="c", subcore_axis_name="s"),
        scratch_types=[
            pltpu.VMEM((NCH, CHUNK), jnp.int32),
            pltpu.VMEM((CHUNK, D), jnp.float32),
            pltpu.SemaphoreType.DMA,
        ],
    )(idx2d, table)


def kernel(clauses, E):
    idx2d = clauses.astype(jnp.int32).reshape(B_TOTAL // CHUNK, CHUNK)
    out = _embedding_gather(idx2d, E)
    return out.reshape(clauses.shape[0], clauses.shape[1], D)


# speedup vs baseline: 1.0330x; 1.0330x over previous
import functools

import jax
import jax.numpy as jnp
from jax import lax
from jax.experimental import pallas as pl
from jax.experimental.pallas import tpu as pltpu
from jax.experimental.pallas import tpu_sc as plsc

VOCAB = 100000
D = 200
B_TOTAL = 4096 * 50

NC = 2
NS = 16
NW = NC * NS

CHUNK = 128
PER_W = B_TOTAL // NW
NCH = PER_W // CHUNK


def _gather_body(idx_hbm, table_hbm, out_hbm, idx_v, rows_v, sem):
    wid = lax.axis_index("s") * NC + lax.axis_index("c")
    pltpu.sync_copy(idx_hbm.at[wid], idx_v)
    out_base = wid * PER_W

    def step(j, carry):
        pltpu.async_copy(table_hbm.at[idx_v.at[j]], rows_v, sem).wait()
        pltpu.sync_copy(rows_v, out_hbm.at[pl.ds(out_base + j * CHUNK, CHUNK)])
        return carry

    lax.fori_loop(0, NCH, step, 0)


@jax.jit
def _embedding_gather(idx2d, table):
    return pl.kernel(
        _gather_body,
        out_type=jax.ShapeDtypeStruct((B_TOTAL, D), jnp.float32),
        mesh=plsc.VectorSubcoreMesh(core_axis_name="c", subcore_axis_name="s"),
        scratch_types=[
            pltpu.VMEM((NCH, CHUNK), jnp.int32),
            pltpu.VMEM((CHUNK, D), jnp.float32),
            pltpu.SemaphoreType.DMA,
        ],
        compiler_params=pltpu.CompilerParams(use_tc_tiling_on_sc=False),
    )(idx2d, table)


def kernel(clauses, E):
    idx2d = clauses.astype(jnp.int32).reshape(NW, NCH, CHUNK)
    out = _embedding_gather(idx2d, E)
    return out.reshape(clauses.shape[0], clauses.shape[1], D)


# speedup vs baseline: 1.0710x; 1.0368x over previous
import functools

import jax
import jax.numpy as jnp
from jax import lax
from jax.experimental import pallas as pl
from jax.experimental.pallas import tpu as pltpu
from jax.experimental.pallas import tpu_sc as plsc

VOCAB = 100000
D = 200
B_TOTAL = 4096 * 50

NC = 2
NS = 16
NW = NC * NS

CHUNK = 128
PER_W = B_TOTAL // NW
NCH = PER_W // CHUNK


NBUF = 2


def _gather_body(idx_hbm, table_hbm, out_hbm, idx_v, rows0, rows1, sem0, sem1):
    wid = lax.axis_index("s") * NC + lax.axis_index("c")
    pltpu.sync_copy(idx_hbm.at[wid], idx_v)
    out_base = wid * PER_W
    bufs = (rows0, rows1)
    sems = (sem0, sem1)

    def start(c, b):
        pltpu.async_copy(table_hbm.at[idx_v.at[c]], bufs[b], sems[b])

    def finish(c, b):
        pltpu.make_async_copy(table_hbm.at[idx_v.at[c]], bufs[b], sems[b]).wait()
        pltpu.sync_copy(bufs[b], out_hbm.at[pl.ds(out_base + c * CHUNK, CHUNK)])

    for b in range(NBUF):
        start(b, b)

    def step(i, carry):
        g = i * NBUF
        for b in range(NBUF):
            c = g + b
            finish(c, b)
            start(c + NBUF, b)
        return carry

    lax.fori_loop(0, NCH // NBUF - 1, step, 0)
    for b in range(NBUF):
        finish(NCH - NBUF + b, b)


@jax.jit
def _embedding_gather(idx2d, table):
    return pl.kernel(
        _gather_body,
        out_type=jax.ShapeDtypeStruct((B_TOTAL, D), jnp.float32),
        mesh=plsc.VectorSubcoreMesh(core_axis_name="c", subcore_axis_name="s"),
        scratch_types=[
            pltpu.VMEM((NCH, CHUNK), jnp.int32),
            pltpu.VMEM((CHUNK, D), jnp.float32),
            pltpu.VMEM((CHUNK, D), jnp.float32),
            pltpu.SemaphoreType.DMA,
            pltpu.SemaphoreType.DMA,
        ],
        compiler_params=pltpu.CompilerParams(use_tc_tiling_on_sc=False),
    )(idx2d, table)


def kernel(clauses, E):
    idx2d = clauses.astype(jnp.int32).reshape(NW, NCH, CHUNK)
    out = _embedding_gather(idx2d, E)
    return out.reshape(clauses.shape[0], clauses.shape[1], D)


# speedup vs baseline: 1.9322x; 1.8041x over previous
import functools

import jax
import jax.numpy as jnp
from jax import lax
from jax.experimental import pallas as pl
from jax.experimental.pallas import tpu as pltpu
from jax.experimental.pallas import tpu_sc as plsc

VOCAB = 100000
D = 200
DH = 128
DT = D - DH
BATCH = 4096
HIST = 50
HPAD = 56

NC = 2
NS = 16
NW = NC * NS
BPW = BATCH // NW
NBUF = 2


def _gather_body(idx_hbm, table_hbm, tail_hbm, out_hbm,
                 idx_v, t0, t1, o0, o1,
                 hs0, hs1, ts0, ts1):
    wid = lax.axis_index("s") * NC + lax.axis_index("c")
    pltpu.sync_copy(idx_hbm.at[pl.ds(wid * BPW * HPAD, BPW * HPAD)], idx_v)
    b0 = wid * BPW
    tbufs = (t0, t1)
    obufs = (o0, o1)
    hsems = (hs0, hs1)
    tsems = (ts0, ts1)
    head_view = table_hbm.at[:, pl.ds(0, DH)]

    def start(bb, b):
        idx = idx_v.at[pl.ds(bb * HPAD, HIST)]
        pltpu.async_copy(head_view.at[idx], obufs[b].at[:, pl.ds(0, DH)],
                         hsems[b])
        pltpu.async_copy(tail_hbm.at[idx], tbufs[b], tsems[b])

    def finish(bb, b):
        idx = idx_v.at[pl.ds(bb * HPAD, HIST)]
        pltpu.make_async_copy(head_view.at[idx], obufs[b].at[:, pl.ds(0, DH)],
                              hsems[b]).wait()
        pltpu.make_async_copy(tail_hbm.at[idx], tbufs[b], tsems[b]).wait()
        ob = obufs[b]

        lanes = DH + 64 + lax.iota(jnp.int32, 16)
        tail8 = lax.iota(jnp.int32, 16) < (DT - 64)

        def row(r, carry):
            tb = tbufs[b]
            for k in range(4):
                ob[r, pl.ds(DH + 16 * k, 16)] = tb[r, pl.ds(16 * k, 16)]
            rows = jnp.full((16,), r, jnp.int32)
            plsc.store_scatter(ob, [rows, lanes], tb[r, pl.ds(64, 16)],
                               mask=tail8)
            return carry

        lax.fori_loop(0, HIST, row, 0)
        pltpu.sync_copy(ob, out_hbm.at[b0 + bb])

    for b in range(NBUF):
        start(b, b)

    def step(i, carry):
        g = i * NBUF
        for b in range(NBUF):
            bb = g + b
            finish(bb, b)
            start(bb + NBUF, b)
        return carry

    lax.fori_loop(0, BPW // NBUF - 1, step, 0)
    for b in range(NBUF):
        finish(BPW - NBUF + b, b)


@jax.jit
def _embedding_gather(idx1d, table, tail):
    return pl.kernel(
        _gather_body,
        out_type=jax.ShapeDtypeStruct((BATCH, HIST, D), jnp.float32),
        mesh=plsc.VectorSubcoreMesh(core_axis_name="c", subcore_axis_name="s"),
        scratch_types=[
            pltpu.VMEM((BPW * HPAD,), jnp.int32),
            pltpu.VMEM((HIST, DH), jnp.float32),
            pltpu.VMEM((HIST, DH), jnp.float32),
            pltpu.VMEM((HIST, D), jnp.float32),
            pltpu.VMEM((HIST, D), jnp.float32),
            pltpu.SemaphoreType.DMA,
            pltpu.SemaphoreType.DMA,
            pltpu.SemaphoreType.DMA,
            pltpu.SemaphoreType.DMA,
        ],
        compiler_params=pltpu.CompilerParams(needs_layout_passes=False),
    )(idx1d, table, tail)


def kernel(clauses, E):
    idx = clauses.astype(jnp.int32)
    idx = jnp.pad(idx, ((0, 0), (0, HPAD - HIST)))
    idx1d = idx.reshape(NW * BPW * HPAD)
    tail = jnp.pad(E[:, DH:], ((0, 0), (0, DH - DT)))
    return _embedding_gather(idx1d, E, tail)
